# Initial kernel scaffold; baseline (speedup 1.0000x reference)
import jax
import jax.numpy as jnp
from jax.experimental import pallas as pl


def kernel(x, Wsel, bsel, Wexp, bexp):
    raise NotImplementedError("write your pallas kernel here")


# speedup vs baseline: 1.3581x; 1.3581x over previous
import functools

import jax
import jax.numpy as jnp
from jax.experimental import pallas as pl
from jax.experimental.pallas import tpu as pltpu

_E = 8
_EPS = 0.2

_SORT_NET = [
    (0, 1), (2, 3), (4, 5), (6, 7),
    (0, 2), (1, 3), (4, 6), (5, 7),
    (1, 2), (5, 6), (0, 4), (3, 7),
    (1, 5), (2, 6),
    (1, 4), (3, 6),
    (2, 4), (3, 5),
    (3, 4),
]


def _routing_body(x_ref, wsel_ref, bsel_ref, weff_ref, cost_ref):
    x = x_ref[...]
    logits = jax.lax.dot_general(
        x, wsel_ref[...], (((1,), (1,)), ((), ())),
        preferred_element_type=jnp.float32) + bsel_ref[...]
    m = jnp.max(logits, axis=1, keepdims=True)
    ex = jnp.exp(logits - m)
    w = ex / jnp.sum(ex, axis=1, keepdims=True)

    ws = [w[:, j:j + 1] for j in range(_E)]
    ids = [jnp.full(ws[0].shape, j, dtype=jnp.int32) for j in range(_E)]
    for a, b in _SORT_NET:
        wa, wb = ws[a], ws[b]
        ia, ib = ids[a], ids[b]
        swap = (wb > wa) | ((wb == wa) & (ib < ia))
        ws[a] = jnp.where(swap, wb, wa)
        ws[b] = jnp.where(swap, wa, wb)
        ids[a] = jnp.where(swap, ib, ia)
        ids[b] = jnp.where(swap, ia, ib)

    cum = [ws[0]]
    for k in range(1, _E):
        cum.append(cum[-1] + ws[k])

    lim = jnp.float32(1.0 - _EPS)
    sp = []
    for k in range(_E):
        nxt = cum[k + 1] if k < _E - 1 else jnp.full(cum[0].shape, 1.0, jnp.float32)
        sp.append(jax.nn.relu(jnp.minimum(nxt, lim) - cum[k]))

    cost = sp[_E - 1]
    for k in range(_E - 1):
        cost = cost + jnp.where(sp[k + 1] > 0, jnp.float32(1.0), sp[k])

    weff_cols = []
    for k in range(_E):
        col = jnp.zeros(cum[0].shape, jnp.float32)
        for j in range(_E):
            col = jnp.where(ids[k] == j, sp[j], col)
        weff_cols.append(col)

    weff_ref[...] = jnp.concatenate(weff_cols, axis=1)
    cost_ref[...] = cost


def _combine_body(weff_ref, x_ref, wexp_ref, bexp_ref, out_ref):
    e = pl.program_id(1)
    d = jax.lax.dot_general(
        x_ref[...], wexp_ref[0], (((1,), (1,)), ((), ())),
        preferred_element_type=jnp.float32)
    weff = weff_ref[...]
    lane = jax.lax.broadcasted_iota(jnp.int32, weff.shape, 1)
    wcol = jnp.sum(jnp.where(lane == e, weff, 0.0), axis=1, keepdims=True)
    contrib = wcol * (d + bexp_ref[0])

    @pl.when(e == 0)
    def _init():
        out_ref[...] = contrib

    @pl.when(e != 0)
    def _acc():
        out_ref[...] += contrib


@jax.jit
def kernel(x, Wsel, bsel, Wexp, bexp):
    n, nin = x.shape
    nout = Wexp.shape[1]
    tb = 512
    n_tb = n // tb

    weff, cost = pl.pallas_call(
        _routing_body,
        grid=(n_tb,),
        in_specs=[
            pl.BlockSpec((tb, nin), lambda t: (t, 0)),
            pl.BlockSpec((_E, nin), lambda t: (0, 0)),
            pl.BlockSpec((1, _E), lambda t: (0, 0)),
        ],
        out_specs=[
            pl.BlockSpec((tb, _E), lambda t: (t, 0)),
            pl.BlockSpec((tb, 1), lambda t: (t, 0)),
        ],
        out_shape=[
            jax.ShapeDtypeStruct((n, _E), jnp.float32),
            jax.ShapeDtypeStruct((n, 1), jnp.float32),
        ],
        compiler_params=pltpu.CompilerParams(
            dimension_semantics=("parallel",)),
    )(x, Wsel, bsel.reshape(1, _E))

    out = pl.pallas_call(
        _combine_body,
        grid=(n_tb, _E),
        in_specs=[
            pl.BlockSpec((tb, _E), lambda t, e: (t, 0)),
            pl.BlockSpec((tb, nin), lambda t, e: (t, 0)),
            pl.BlockSpec((1, nout, nin), lambda t, e: (e, 0, 0)),
            pl.BlockSpec((1, 1, nout), lambda t, e: (e, 0, 0)),
        ],
        out_specs=pl.BlockSpec((tb, nout), lambda t, e: (t, 0)),
        out_shape=jax.ShapeDtypeStruct((n, nout), jnp.float32),
        compiler_params=pltpu.CompilerParams(
            dimension_semantics=("parallel", "arbitrary")),
    )(weff, x, Wexp, bexp.reshape(_E, 1, nout))

    return (out, cost.reshape(n))


# speedup vs baseline: 1.7783x; 1.3094x over previous
import functools

import jax
import jax.numpy as jnp
from jax.experimental import pallas as pl
from jax.experimental.pallas import tpu as pltpu

_E = 8
_EPS = 0.2

_SORT_NET = [
    (0, 1), (2, 3), (4, 5), (6, 7),
    (0, 2), (1, 3), (4, 6), (5, 7),
    (1, 2), (5, 6), (0, 4), (3, 7),
    (1, 5), (2, 6),
    (1, 4), (3, 6),
    (2, 4), (3, 5),
    (3, 4),
]


def _routing_body(x_ref, wsel_ref, bsel_ref, weff_ref, cost_ref):
    x = x_ref[...]
    logits = jax.lax.dot_general(
        x, wsel_ref[...], (((1,), (1,)), ((), ())),
        preferred_element_type=jnp.float32) + bsel_ref[...]
    m = jnp.max(logits, axis=1, keepdims=True)
    ex = jnp.exp(logits - m)
    w = ex / jnp.sum(ex, axis=1, keepdims=True)

    ws = [w[:, j:j + 1] for j in range(_E)]
    ids = [jnp.full(ws[0].shape, j, dtype=jnp.int32) for j in range(_E)]
    for a, b in _SORT_NET:
        wa, wb = ws[a], ws[b]
        ia, ib = ids[a], ids[b]
        swap = (wb > wa) | ((wb == wa) & (ib < ia))
        ws[a] = jnp.where(swap, wb, wa)
        ws[b] = jnp.where(swap, wa, wb)
        ids[a] = jnp.where(swap, ib, ia)
        ids[b] = jnp.where(swap, ia, ib)

    cum = [ws[0]]
    for k in range(1, _E):
        cum.append(cum[-1] + ws[k])

    lim = jnp.float32(1.0 - _EPS)
    sp = []
    for k in range(_E):
        nxt = cum[k + 1] if k < _E - 1 else jnp.full(cum[0].shape, 1.0, jnp.float32)
        sp.append(jax.nn.relu(jnp.minimum(nxt, lim) - cum[k]))

    cost = sp[_E - 1]
    for k in range(_E - 1):
        cost = cost + jnp.where(sp[k + 1] > 0, jnp.float32(1.0), sp[k])

    weff_cols = []
    for k in range(_E):
        col = jnp.zeros(cum[0].shape, jnp.float32)
        for j in range(_E):
            col = jnp.where(ids[k] == j, sp[j], col)
        weff_cols.append(col)

    weff_ref[...] = jnp.concatenate(weff_cols, axis=1)
    cost_ref[...] = cost


def _combine_body(weff_ref, x_ref, wexp_ref, bexp_ref, out_ref):
    e = pl.program_id(1)
    d = jax.lax.dot_general(
        x_ref[...], wexp_ref[0], (((1,), (1,)), ((), ())),
        preferred_element_type=jnp.float32)
    weff = weff_ref[...]
    lane = jax.lax.broadcasted_iota(jnp.int32, weff.shape, 1)
    wcol = jnp.sum(jnp.where(lane == e, weff, 0.0), axis=1, keepdims=True)
    contrib = wcol * (d + bexp_ref[0])

    @pl.when(e == 0)
    def _init():
        out_ref[...] = contrib

    @pl.when(e != 0)
    def _acc():
        out_ref[...] += contrib


@jax.jit
def kernel(x, Wsel, bsel, Wexp, bexp):
    n, nin = x.shape
    nout = Wexp.shape[1]
    tb = 512
    n_tb = n // tb

    weff, cost = pl.pallas_call(
        _routing_body,
        grid=(n_tb,),
        in_specs=[
            pl.BlockSpec((tb, nin), lambda t: (t, 0)),
            pl.BlockSpec((_E, nin), lambda t: (0, 0)),
            pl.BlockSpec((1, _E), lambda t: (0, 0)),
        ],
        out_specs=[
            pl.BlockSpec((tb, _E), lambda t: (t, 0)),
            pl.BlockSpec((tb, 1), lambda t: (t, 0)),
        ],
        out_shape=[
            jax.ShapeDtypeStruct((n, _E), jnp.float32),
            jax.ShapeDtypeStruct((n, 1), jnp.float32),
        ],
        compiler_params=pltpu.CompilerParams(
            dimension_semantics=("parallel",)),
    )(x, Wsel, bsel.reshape(1, _E))

    ctb = 2048
    n_ctb = n // ctb
    out = pl.pallas_call(
        _combine_body,
        grid=(n_ctb, _E),
        in_specs=[
            pl.BlockSpec((ctb, _E), lambda t, e: (t, 0)),
            pl.BlockSpec((ctb, nin), lambda t, e: (t, 0)),
            pl.BlockSpec((1, nout, nin), lambda t, e: (e, 0, 0)),
            pl.BlockSpec((1, 1, nout), lambda t, e: (e, 0, 0)),
        ],
        out_specs=pl.BlockSpec((ctb, nout), lambda t, e: (t, 0)),
        out_shape=jax.ShapeDtypeStruct((n, nout), jnp.float32),
        compiler_params=pltpu.CompilerParams(
            dimension_semantics=("parallel", "arbitrary")),
    )(weff, x, Wexp, bexp.reshape(_E, 1, nout))

    return (out, cost.reshape(n))
